# baseline (speedup 1.0000x reference)
import functools
import math

import jax
import jax.numpy as jnp
from jax import lax
from jax.experimental import pallas as pl
from jax.experimental.pallas import tpu as pltpu
from jax.experimental.pallas import tpu_sc as plsc

_SIZE = 32000
_PAD = 0
_SMOOTHING = 0.1
_CONF = 1.0 - _SMOOTHING
_EPS = _SMOOTHING / (_SIZE - 2)
_C = (_SIZE - 2) * _EPS * math.log(_EPS) + _CONF * math.log(_CONF)

_BR = 64

_info = plsc.get_sparse_core_info()
_NC, _NS, _L = _info.num_cores, _info.num_subcores, _info.num_lanes
_NW = _NC * _NS


def _acc_scalar(o_ref, i, partial):
    @pl.when(i == 0)
    def _init():
        o_ref[...] = jnp.zeros_like(o_ref)

    o_ref[...] += jnp.full((1, 1), 1.0, jnp.float32) * partial


def _tc_dense_kernel(t_ref, x_ref, o_ref):
    i = pl.program_id(0)
    x = x_ref[...]
    t = t_ref[0, 0, :]
    m = (t != _PAD).astype(jnp.float32)
    rowsum = jnp.sum(x, axis=1)
    col0 = x[:, 0]
    partial = (-_EPS) * jnp.sum(rowsum * m) + _EPS * jnp.sum(col0 * m) \
        + _C * jnp.sum(m)
    _acc_scalar(o_ref, i, partial)


def _tc_full_kernel(t_ref, x_ref, o_ref):
    i = pl.program_id(0)
    x = x_ref[...]
    t = t_ref[0, 0, :]
    m = (t != _PAD).astype(jnp.float32)
    cols = jax.lax.broadcasted_iota(jnp.int32, x.shape, 1)
    z = jnp.where(cols == t[:, None], (_CONF / _EPS) * x, x)
    zsum = jnp.sum(z, axis=1)
    col0 = x[:, 0]
    partial = (-_EPS) * jnp.sum(zsum * m) + _EPS * jnp.sum(col0 * m) \
        + _C * jnp.sum(m)
    _acc_scalar(o_ref, i, partial)


def _tc_part(x, target, body):
    n, size = x.shape
    nb = n // _BR
    t3 = target.reshape(nb, 1, _BR)
    out = pl.pallas_call(
        body,
        grid=(nb,),
        in_specs=[
            pl.BlockSpec((1, 1, _BR), lambda i: (i, 0, 0)),
            pl.BlockSpec((_BR, size), lambda i: (i, 0)),
        ],
        out_specs=pl.BlockSpec((1, 1), lambda i: (0, 0)),
        out_shape=jax.ShapeDtypeStruct((1, 1), jnp.float32),
    )(t3, x)
    return out[0, 0]


def _make_sc_gather(n):
    bpw = n // _NW
    mesh = plsc.VectorSubcoreMesh(core_axis_name="c", subcore_axis_name="s")

    chunk = 32

    @functools.partial(
        pl.kernel,
        mesh=mesh,
        out_type=jax.ShapeDtypeStruct((_NW, 128), jnp.float32),
        scratch_types=[
            pltpu.VMEM((bpw,), jnp.int32),
            pltpu.VMEM((chunk, 8, 128), jnp.float32),
            pltpu.VMEM((128,), jnp.float32),
            pltpu.SemaphoreType.DMA,
        ],
    )
    def _sc(x_hbm, t_hbm, out_hbm, t_v, val_v, acc_v, sem):
        wid = lax.axis_index("s") * _NC + lax.axis_index("c")
        base = pl.multiple_of(wid * bpw, bpw)
        pltpu.sync_copy(t_hbm.at[pl.ds(base, bpw)], t_v)
        iota16 = lax.iota(jnp.int32, _L)
        acc = jnp.zeros((_L,), jnp.float32)
        for c0 in range(0, bpw, chunk):
            descs = []
            tregs = []
            for j in range(chunk // _L):
                t16 = t_v[pl.ds((c0 + j * _L), _L)]
                tregs.append(t16)
                for k in range(_L):
                    i = c0 + j * _L + k
                    t_i = t16[k]
                    cb = pl.multiple_of((t_i >> 7) << 7, 128)
                    r8 = pl.multiple_of(base + (i // 8) * 8, 8)
                    descs.append(
                        pltpu.async_copy(
                            x_hbm.at[pl.ds(r8, 8), pl.ds(cb, 128)],
                            val_v.at[i - c0], sem))
            for d in descs:
                d.wait()
            for j in range(chunk // _L):
                t16 = tregs[j]
                for k in range(_L):
                    i = c0 + j * _L + k
                    t_i = t16[k]
                    co = pl.multiple_of(((t_i & 127) >> 4) << 4, _L)
                    v16 = val_v[i - c0, i % 8, pl.ds(co, _L)]
                    lane = jnp.where(t_i != _PAD, t_i & (_L - 1), _L)
                    acc = acc + jnp.where(iota16 == lane, v16, 0.0)
        acc_v[pl.ds(0, _L)] = acc
        for j in range(1, 128 // _L):
            acc_v[pl.ds(j * _L, _L)] = jnp.zeros((_L,), jnp.float32)
        pltpu.sync_copy(acc_v, out_hbm.at[wid])

    return _sc


def kernel(x, target):
    return _tc_part(x, target, _tc_full_kernel)


# speedup vs baseline: 1.0609x; 1.0609x over previous
import functools
import math

import jax
import jax.numpy as jnp
from jax import lax
from jax.experimental import pallas as pl
from jax.experimental.pallas import tpu as pltpu
from jax.experimental.pallas import tpu_sc as plsc

_SIZE = 32000
_PAD = 0
_SMOOTHING = 0.1
_CONF = 1.0 - _SMOOTHING
_EPS = _SMOOTHING / (_SIZE - 2)
_C = (_SIZE - 2) * _EPS * math.log(_EPS) + _CONF * math.log(_CONF)

_BR = 128

_info = plsc.get_sparse_core_info()
_NC, _NS, _L = _info.num_cores, _info.num_subcores, _info.num_lanes
_NW = _NC * _NS


def _acc_scalar(o_ref, i, partial):
    @pl.when(i == 0)
    def _init():
        o_ref[...] = jnp.zeros_like(o_ref)

    o_ref[...] += jnp.full((1, 1), 1.0, jnp.float32) * partial


def _tc_dense_kernel(t_ref, x_ref, o_ref):
    i = pl.program_id(0)
    x = x_ref[...]
    t = t_ref[0, 0, :]
    m = (t != _PAD).astype(jnp.float32)
    rowsum = jnp.sum(x, axis=1)
    col0 = x[:, 0]
    partial = (-_EPS) * jnp.sum(rowsum * m) + _EPS * jnp.sum(col0 * m) \
        + _C * jnp.sum(m)
    _acc_scalar(o_ref, i, partial)


def _tc_full_kernel(t_ref, x_ref, o_ref):
    i = pl.program_id(0)
    x = x_ref[...]
    t = t_ref[0, 0, :]
    m = (t != _PAD).astype(jnp.float32)
    cols = jax.lax.broadcasted_iota(jnp.int32, x.shape, 1)
    z = jnp.where(cols == t[:, None], (_CONF / _EPS) * x, x)
    zsum = jnp.sum(z, axis=1)
    col0 = x[:, 0]
    partial = (-_EPS) * jnp.sum(zsum * m) + _EPS * jnp.sum(col0 * m) \
        + _C * jnp.sum(m)
    _acc_scalar(o_ref, i, partial)


def _tc_part(x, target, body):
    n, size = x.shape
    nb = n // _BR
    t3 = target.reshape(nb, 1, _BR)
    out = pl.pallas_call(
        body,
        grid=(nb,),
        in_specs=[
            pl.BlockSpec((1, 1, _BR), lambda i: (i, 0, 0)),
            pl.BlockSpec((_BR, size), lambda i: (i, 0)),
        ],
        out_specs=pl.BlockSpec((1, 1), lambda i: (0, 0)),
        out_shape=jax.ShapeDtypeStruct((1, 1), jnp.float32),
    )(t3, x)
    return out[0, 0]


_BR2 = 256
_BC2 = 16000


def _tc_full2_kernel(t_ref, x_ref, o_ref):
    i = pl.program_id(0)
    j = pl.program_id(1)
    x = x_ref[...]
    t = t_ref[0, 0, :]
    m = (t != _PAD).astype(jnp.float32)
    cols = jax.lax.broadcasted_iota(jnp.int32, x.shape, 1)
    tq = t[:, None] - j * _BC2
    z = jnp.where(cols == tq, (_CONF / _EPS) * x, x)
    zsum = jnp.sum(z, axis=1)
    partial = (-_EPS) * jnp.sum(zsum * m)
    col0 = x[:, 0]
    extra = _EPS * jnp.sum(col0 * m) + _C * jnp.sum(m)
    partial = partial + jnp.where(j == 0, extra, 0.0)
    _acc_scalar(o_ref, (i + j), partial)


def _tc_part2(x, target):
    n, size = x.shape
    nb = n // _BR2
    nc = size // _BC2
    t3 = target.reshape(nb, 1, _BR2)
    out = pl.pallas_call(
        _tc_full2_kernel,
        grid=(nb, nc),
        in_specs=[
            pl.BlockSpec((1, 1, _BR2), lambda i, j: (i, 0, 0)),
            pl.BlockSpec((_BR2, _BC2), lambda i, j: (i, j)),
        ],
        out_specs=pl.BlockSpec((1, 1), lambda i, j: (0, 0)),
        out_shape=jax.ShapeDtypeStruct((1, 1), jnp.float32),
    )(t3, x)
    return out[0, 0]


def _make_sc_gather(n):
    bpw = n // _NW
    mesh = plsc.VectorSubcoreMesh(core_axis_name="c", subcore_axis_name="s")

    chunk = 32

    @functools.partial(
        pl.kernel,
        mesh=mesh,
        out_type=jax.ShapeDtypeStruct((_NW, 128), jnp.float32),
        scratch_types=[
            pltpu.VMEM((bpw,), jnp.int32),
            pltpu.VMEM((chunk, 8, 128), jnp.float32),
            pltpu.VMEM((128,), jnp.float32),
            pltpu.SemaphoreType.DMA,
        ],
    )
    def _sc(x_hbm, t_hbm, out_hbm, t_v, val_v, acc_v, sem):
        wid = lax.axis_index("s") * _NC + lax.axis_index("c")
        base = pl.multiple_of(wid * bpw, bpw)
        pltpu.sync_copy(t_hbm.at[pl.ds(base, bpw)], t_v)
        iota16 = lax.iota(jnp.int32, _L)
        acc = jnp.zeros((_L,), jnp.float32)
        for c0 in range(0, bpw, chunk):
            descs = []
            tregs = []
            for j in range(chunk // _L):
                t16 = t_v[pl.ds((c0 + j * _L), _L)]
                tregs.append(t16)
                for k in range(_L):
                    i = c0 + j * _L + k
                    t_i = t16[k]
                    cb = pl.multiple_of((t_i >> 7) << 7, 128)
                    r8 = pl.multiple_of(base + (i // 8) * 8, 8)
                    descs.append(
                        pltpu.async_copy(
                            x_hbm.at[pl.ds(r8, 8), pl.ds(cb, 128)],
                            val_v.at[i - c0], sem))
            for d in descs:
                d.wait()
            for j in range(chunk // _L):
                t16 = tregs[j]
                for k in range(_L):
                    i = c0 + j * _L + k
                    t_i = t16[k]
                    co = pl.multiple_of(((t_i & 127) >> 4) << 4, _L)
                    v16 = val_v[i - c0, i % 8, pl.ds(co, _L)]
                    lane = jnp.where(t_i != _PAD, t_i & (_L - 1), _L)
                    acc = acc + jnp.where(iota16 == lane, v16, 0.0)
        acc_v[pl.ds(0, _L)] = acc
        for j in range(1, 128 // _L):
            acc_v[pl.ds(j * _L, _L)] = jnp.zeros((_L,), jnp.float32)
        pltpu.sync_copy(acc_v, out_hbm.at[wid])

    return _sc


def kernel(x, target):
    return _tc_part2(x, target)


# speedup vs baseline: 1.0630x; 1.0020x over previous
import math

import jax
import jax.numpy as jnp
from jax.experimental import pallas as pl

_PAD = 0
_SMOOTHING = 0.1
_CONF = 1.0 - _SMOOTHING

_BR = 128


def _loss_kernel(eps, c_const, t_ref, x_ref, o_ref):
    i = pl.program_id(0)
    x = x_ref[...]
    t = t_ref[0, 0, :]
    m = (t != _PAD).astype(jnp.float32)
    cols = jax.lax.broadcasted_iota(jnp.int32, x.shape, 1)
    z = jnp.where(cols == t[:, None], (_CONF / eps) * x, x)
    zsum = jnp.sum(z, axis=1)
    col0 = x[:, 0]
    partial = (-eps) * jnp.sum(zsum * m) + eps * jnp.sum(col0 * m) \
        + c_const * jnp.sum(m)

    @pl.when(i == 0)
    def _init():
        o_ref[...] = jnp.zeros_like(o_ref)

    o_ref[...] += jnp.full((1, 1), 1.0, jnp.float32) * partial


def kernel(x, target):
    n, size = x.shape
    eps = _SMOOTHING / (size - 2)
    c_const = (size - 2) * eps * math.log(eps) + _CONF * math.log(_CONF)
    nb = n // _BR
    t3 = target.reshape(nb, 1, _BR)
    out = pl.pallas_call(
        lambda t_ref, x_ref, o_ref: _loss_kernel(eps, c_const,
                                                 t_ref, x_ref, o_ref),
        grid=(nb,),
        in_specs=[
            pl.BlockSpec((1, 1, _BR), lambda i: (i, 0, 0)),
            pl.BlockSpec((_BR, size), lambda i: (i, 0)),
        ],
        out_specs=pl.BlockSpec((1, 1), lambda i: (0, 0)),
        out_shape=jax.ShapeDtypeStruct((1, 1), jnp.float32),
    )(t3, x)
    return out[0, 0]
